# baseline (speedup 1.0000x reference)
import functools

import jax
import jax.numpy as jnp
from jax import lax
from jax.experimental import pallas as pl
from jax.experimental.pallas import tpu as pltpu
from jax.experimental.pallas import tpu_sc as plsc

B = 4
P = 512
T = 2048
DI = 1024
DO = 1024
L = 16
R = B * P
YROWS = (L + 1) * R


def _tc_body(pi_ref, x_ref, w_ref, y_ref, idx_ref):
    l = pl.program_id(0)

    @pl.when(l < L)
    def _matmul():
        y_ref[...] = lax.dot_general(
            x_ref[...], w_ref[0],
            dimension_numbers=(((1,), (1,)), ((), ())),
            preferred_element_type=jnp.float32,
        )

    @pl.when(l == L)
    def _zeros():
        y_ref[...] = jnp.zeros_like(y_ref)

    @pl.when(l == 0)
    def _rowmap():
        pi = pi_ref[0, :]
        piB = jnp.broadcast_to(pi[None, :], (T, P))
        t2 = lax.broadcasted_iota(jnp.int32, (T, P), 0)
        lt = piB < t2
        idx = jnp.sum(lt.astype(jnp.int32), axis=1)
        prev = jnp.max(jnp.where(lt, piB, -1), axis=1)
        t1 = lax.iota(jnp.int32, T)
        pos = jnp.minimum(t1 - 1 - prev, L - 1)
        valid = idx < P
        l_eff = jnp.where(valid, pos, L)
        j_eff = jnp.where(valid, idx, 0)
        rows = l_eff * R + j_eff
        boff = lax.broadcasted_iota(jnp.int32, (B, T), 0) * P
        idx_ref[...] = rows[None, :] + boff


def _tc_call(pi2d, x_flat, W):
    return pl.pallas_call(
        _tc_body,
        grid=(L + 1,),
        in_specs=[
            pl.BlockSpec((1, P), lambda l: (0, 0)),
            pl.BlockSpec((R, DI), lambda l: (0, 0)),
            pl.BlockSpec((1, DO, DI), lambda l: (jnp.minimum(l, L - 1), 0, 0)),
        ],
        out_specs=[
            pl.BlockSpec((R, DO), lambda l: (l, 0)),
            pl.BlockSpec((B, T), lambda l: (0, 0)),
        ],
        out_shape=[
            jax.ShapeDtypeStruct((YROWS, DO), jnp.float32),
            jax.ShapeDtypeStruct((B, T), jnp.int32),
        ],
    )(pi2d, x_flat, W)


NW = 32
RPW = (B * T) // NW
CH = 32
NCHUNK = RPW // CH


def _sc_body(y_hbm, idx_hbm, out_hbm, idx_v, rows_v, sem_g, sem_s):
    wid = lax.axis_index("s") * 2 + lax.axis_index("c")
    base = wid * RPW
    pltpu.sync_copy(idx_hbm.at[pl.ds(base, RPW)], idx_v)
    pltpu.async_copy(y_hbm.at[idx_v.at[pl.ds(0, CH)]], rows_v.at[0], sem_g)
    for c in range(NCHUNK):
        pltpu.make_async_copy(y_hbm.at[idx_v.at[pl.ds(c * CH, CH)]],
                              rows_v.at[c % 2], sem_g).wait()
        if c + 1 < NCHUNK:
            pltpu.async_copy(
                y_hbm.at[idx_v.at[pl.ds((c + 1) * CH, CH)]],
                rows_v.at[(c + 1) % 2], sem_g)
        if c > 0:
            pltpu.make_async_copy(
                rows_v.at[(c - 1) % 2],
                out_hbm.at[pl.ds(base + (c - 1) * CH, CH)], sem_s).wait()
        pltpu.async_copy(rows_v.at[c % 2],
                         out_hbm.at[pl.ds(base + c * CH, CH)], sem_s)
    pltpu.make_async_copy(rows_v.at[(NCHUNK - 1) % 2],
                          out_hbm.at[pl.ds(base + (NCHUNK - 1) * CH, CH)],
                          sem_s).wait()


def _sc_call(y, idx_flat):
    mesh = plsc.VectorSubcoreMesh(core_axis_name="c", subcore_axis_name="s")
    f = pl.kernel(
        _sc_body,
        out_type=jax.ShapeDtypeStruct((B * T, DO), jnp.float32),
        mesh=mesh,
        scratch_types=[
            pltpu.VMEM((RPW,), jnp.int32),
            pltpu.VMEM((2, CH, DO), jnp.float32),
            pltpu.SemaphoreType.DMA,
            pltpu.SemaphoreType.DMA,
        ],
    )
    return f(y, idx_flat)


def kernel(pooled_vectors, pooling_indices, target_length, W):
    x_flat = pooled_vectors.reshape(R, DI).astype(jnp.bfloat16)
    W = W.astype(jnp.bfloat16)
    pi2d = pooling_indices.reshape(1, P).astype(jnp.int32)
    y, row_idx = _tc_call(pi2d, x_flat, W)
    out = _sc_call(y, row_idx.reshape(B * T))
    return out.reshape(B, T, DO)


# speedup vs baseline: 1.0263x; 1.0263x over previous
import functools

import jax
import jax.numpy as jnp
from jax import lax
from jax.experimental import pallas as pl
from jax.experimental.pallas import tpu as pltpu
from jax.experimental.pallas import tpu_sc as plsc

B = 4
P = 512
T = 2048
DI = 1024
DO = 1024
L = 16
R = B * P
YROWS = (L + 1) * R


def _tc_body(pi_ref, x_ref, w_ref, y_ref, idx_ref, xb_ref):
    l = pl.program_id(0)

    @pl.when(l == 0)
    def _cast_x():
        xb_ref[...] = x_ref[...].astype(jnp.bfloat16)

    @pl.when(l < L)
    def _matmul():
        y_ref[...] = lax.dot_general(
            xb_ref[...], w_ref[0],
            dimension_numbers=(((1,), (1,)), ((), ())),
            preferred_element_type=jnp.float32,
        )

    @pl.when(l == L)
    def _zeros():
        y_ref[...] = jnp.zeros_like(y_ref)

    @pl.when(l == 0)
    def _rowmap():
        pi = pi_ref[0, :]
        piB = jnp.broadcast_to(pi[None, :], (T, P))
        t2 = lax.broadcasted_iota(jnp.int32, (T, P), 0)
        lt = piB < t2
        idx = jnp.sum(lt.astype(jnp.int32), axis=1)
        prev = jnp.max(jnp.where(lt, piB, -1), axis=1)
        t1 = lax.iota(jnp.int32, T)
        pos = jnp.minimum(t1 - 1 - prev, L - 1)
        valid = idx < P
        l_eff = jnp.where(valid, pos, L)
        j_eff = jnp.where(valid, idx, 0)
        rows = l_eff * R + j_eff
        boff = lax.broadcasted_iota(jnp.int32, (B, T), 0) * P
        idx_ref[...] = rows[None, :] + boff


def _tc_call(pi2d, x_flat, W):
    return pl.pallas_call(
        _tc_body,
        grid=(L + 1,),
        in_specs=[
            pl.BlockSpec((1, P), lambda l: (0, 0)),
            pl.BlockSpec((R, DI), lambda l: (0, 0)),
            pl.BlockSpec((1, DO, DI), lambda l: (jnp.minimum(l, L - 1), 0, 0)),
        ],
        out_specs=[
            pl.BlockSpec((R, DO), lambda l: (l, 0)),
            pl.BlockSpec((B, T), lambda l: (0, 0)),
        ],
        out_shape=[
            jax.ShapeDtypeStruct((YROWS, DO), jnp.float32),
            jax.ShapeDtypeStruct((B, T), jnp.int32),
        ],
        scratch_shapes=[pltpu.VMEM((R, DI), jnp.bfloat16)],
    )(pi2d, x_flat, W)


NW = 32
RPW = (B * T) // NW
CH = 32
NCHUNK = RPW // CH


def _sc_body(y_hbm, idx_hbm, out_hbm, idx_v, rows_v, sem_g, sem_s):
    wid = lax.axis_index("s") * 2 + lax.axis_index("c")
    base = wid * RPW
    pltpu.sync_copy(idx_hbm.at[pl.ds(base, RPW)], idx_v)
    pltpu.async_copy(y_hbm.at[idx_v.at[pl.ds(0, CH)]], rows_v.at[0], sem_g)
    for c in range(NCHUNK):
        pltpu.make_async_copy(y_hbm.at[idx_v.at[pl.ds(c * CH, CH)]],
                              rows_v.at[c % 2], sem_g).wait()
        if c + 1 < NCHUNK:
            pltpu.async_copy(
                y_hbm.at[idx_v.at[pl.ds((c + 1) * CH, CH)]],
                rows_v.at[(c + 1) % 2], sem_g)
        if c > 0:
            pltpu.make_async_copy(
                rows_v.at[(c - 1) % 2],
                out_hbm.at[pl.ds(base + (c - 1) * CH, CH)], sem_s).wait()
        pltpu.async_copy(rows_v.at[c % 2],
                         out_hbm.at[pl.ds(base + c * CH, CH)], sem_s)
    pltpu.make_async_copy(rows_v.at[(NCHUNK - 1) % 2],
                          out_hbm.at[pl.ds(base + (NCHUNK - 1) * CH, CH)],
                          sem_s).wait()


def _sc_call(y, idx_flat):
    mesh = plsc.VectorSubcoreMesh(core_axis_name="c", subcore_axis_name="s")
    f = pl.kernel(
        _sc_body,
        out_type=jax.ShapeDtypeStruct((B * T, DO), jnp.float32),
        mesh=mesh,
        scratch_types=[
            pltpu.VMEM((RPW,), jnp.int32),
            pltpu.VMEM((2, CH, DO), jnp.float32),
            pltpu.SemaphoreType.DMA,
            pltpu.SemaphoreType.DMA,
        ],
    )
    return f(y, idx_flat)


def kernel(pooled_vectors, pooling_indices, target_length, W):
    x_flat = pooled_vectors.reshape(R, DI)
    pi2d = pooling_indices.reshape(1, P).astype(jnp.int32)
    y, row_idx = _tc_call(pi2d, x_flat, W.astype(jnp.bfloat16))
    out = _sc_call(y, row_idx.reshape(B * T))
    return out.reshape(B, T, DO)


# speedup vs baseline: 1.2519x; 1.2197x over previous
import functools

import jax
import jax.numpy as jnp
from jax import lax
from jax.experimental import pallas as pl
from jax.experimental.pallas import tpu as pltpu
from jax.experimental.pallas import tpu_sc as plsc

B = 4
P = 512
T = 2048
DI = 1024
DO = 1024
L = 16
R = B * P
YROWS = (L + 1) * R


def _map_body(pi_ref, idx_ref):
    pi = pi_ref[0, :]
    piB = jnp.broadcast_to(pi[None, :], (T, P))
    t2 = lax.broadcasted_iota(jnp.int32, (T, P), 0)
    lt = piB < t2
    idx = jnp.sum(lt.astype(jnp.int32), axis=1)
    prev = jnp.max(jnp.where(lt, piB, -1), axis=1)
    t1 = lax.iota(jnp.int32, T)
    pos = jnp.minimum(t1 - 1 - prev, L - 1)
    valid = idx < P
    l_eff = jnp.where(valid, pos, L)
    j_eff = jnp.where(valid, idx, 0)
    rows = l_eff * R + j_eff
    boff = lax.broadcasted_iota(jnp.int32, (B, T), 0) * P
    idx_ref[...] = rows[None, :] + boff


def _map_call(pi2d):
    return pl.pallas_call(
        _map_body,
        out_shape=jax.ShapeDtypeStruct((B, T), jnp.int32),
    )(pi2d)


def _mm_body(x_ref, w_ref, y_ref):
    l = pl.program_id(0)

    @pl.when(l < L)
    def _matmul():
        y_ref[...] = lax.dot_general(
            x_ref[...], w_ref[0],
            dimension_numbers=(((1,), (1,)), ((), ())),
            preferred_element_type=jnp.float32,
        )

    @pl.when(l == L)
    def _zeros():
        y_ref[...] = jnp.zeros_like(y_ref)


def _mm_call(x_flat, W):
    return pl.pallas_call(
        _mm_body,
        grid=(L + 1,),
        in_specs=[
            pl.BlockSpec((R, DI), lambda l: (0, 0)),
            pl.BlockSpec((1, DO, DI), lambda l: (jnp.minimum(l, L - 1), 0, 0)),
        ],
        out_specs=pl.BlockSpec((R, DO), lambda l: (l, 0)),
        out_shape=jax.ShapeDtypeStruct((YROWS, DO), jnp.float32),
    )(x_flat, W)


NW = 32
RPW = (B * T) // NW
CH = 32
NCHUNK = RPW // CH


def _sc_body(y_hbm, idx_hbm, out_hbm, idx_v, rows_v, sem_g, sem_s):
    wid = lax.axis_index("s") * 2 + lax.axis_index("c")
    base = wid * RPW
    pltpu.sync_copy(idx_hbm.at[pl.ds(base, RPW)], idx_v)
    pltpu.async_copy(y_hbm.at[idx_v.at[pl.ds(0, CH)]], rows_v.at[0], sem_g)
    for c in range(NCHUNK):
        pltpu.make_async_copy(y_hbm.at[idx_v.at[pl.ds(c * CH, CH)]],
                              rows_v.at[c % 2], sem_g).wait()
        if c + 1 < NCHUNK:
            pltpu.async_copy(
                y_hbm.at[idx_v.at[pl.ds((c + 1) * CH, CH)]],
                rows_v.at[(c + 1) % 2], sem_g)
        if c > 0:
            pltpu.make_async_copy(
                rows_v.at[(c - 1) % 2],
                out_hbm.at[pl.ds(base + (c - 1) * CH, CH)], sem_s).wait()
        pltpu.async_copy(rows_v.at[c % 2],
                         out_hbm.at[pl.ds(base + c * CH, CH)], sem_s)
    pltpu.make_async_copy(rows_v.at[(NCHUNK - 1) % 2],
                          out_hbm.at[pl.ds(base + (NCHUNK - 1) * CH, CH)],
                          sem_s).wait()


def _sc_call(y, idx_flat):
    mesh = plsc.VectorSubcoreMesh(core_axis_name="c", subcore_axis_name="s")
    f = pl.kernel(
        _sc_body,
        out_type=jax.ShapeDtypeStruct((B * T, DO), jnp.float32),
        mesh=mesh,
        scratch_types=[
            pltpu.VMEM((RPW,), jnp.int32),
            pltpu.VMEM((2, CH, DO), jnp.float32),
            pltpu.SemaphoreType.DMA,
            pltpu.SemaphoreType.DMA,
        ],
    )
    return f(y, idx_flat)


def kernel(pooled_vectors, pooling_indices, target_length, W):
    x_flat = pooled_vectors.reshape(R, DI)
    pi2d = pooling_indices.reshape(1, P).astype(jnp.int32)
    row_idx = _map_call(pi2d)
    y = _mm_call(x_flat, W)
    out = _sc_call(y, row_idx.reshape(B * T))
    return out.reshape(B, T, DO)


# speedup vs baseline: 1.2866x; 1.0278x over previous
import functools

import jax
import jax.numpy as jnp
from jax import lax
from jax.experimental import pallas as pl
from jax.experimental.pallas import tpu as pltpu
from jax.experimental.pallas import tpu_sc as plsc

B = 4
P = 512
T = 2048
DI = 1024
DO = 1024
L = 16
R = B * P
YROWS = (L + 1) * R


def _map_body(pi_ref, idx_ref):
    pi = pi_ref[0, :]
    piB = jnp.broadcast_to(pi[None, :], (T, P))
    t2 = lax.broadcasted_iota(jnp.int32, (T, P), 0)
    lt = piB < t2
    idx = jnp.sum(lt.astype(jnp.int32), axis=1)
    prev = jnp.max(jnp.where(lt, piB, -1), axis=1)
    t1 = lax.iota(jnp.int32, T)
    pos = jnp.minimum(t1 - 1 - prev, L - 1)
    valid = idx < P
    l_eff = jnp.where(valid, pos, L)
    j_eff = jnp.where(valid, idx, 0)
    rows = l_eff * R + j_eff
    boff = lax.broadcasted_iota(jnp.int32, (B, T), 0) * P
    idx_ref[...] = rows[None, :] + boff


def _map_call(pi2d):
    return pl.pallas_call(
        _map_body,
        out_shape=jax.ShapeDtypeStruct((B, T), jnp.int32),
    )(pi2d)


def _mm_body(x_ref, w_ref, y_ref):
    l = pl.program_id(0)

    @pl.when(l < L)
    def _matmul():
        y_ref[...] = lax.dot_general(
            x_ref[...], w_ref[0],
            dimension_numbers=(((1,), (1,)), ((), ())),
            preferred_element_type=jnp.float32,
        )

    @pl.when(l == L)
    def _zeros():
        y_ref[...] = jnp.zeros_like(y_ref)


def _mm_call(x_flat, W):
    return pl.pallas_call(
        _mm_body,
        grid=(L + 1,),
        in_specs=[
            pl.BlockSpec((R, DI), lambda l: (0, 0)),
            pl.BlockSpec((1, DO, DI), lambda l: (jnp.minimum(l, L - 1), 0, 0)),
        ],
        out_specs=pl.BlockSpec((R, DO), lambda l: (l, 0)),
        out_shape=jax.ShapeDtypeStruct((YROWS, DO), jnp.float32),
    )(x_flat, W)


NW = 32
RPW = (B * T) // NW
CH = 32
NCHUNK = RPW // CH


NBUF = 3


def _sc_body(y_hbm, idx_hbm, out_hbm, idx_v, rows_v, sem_g, sem_s):
    wid = lax.axis_index("s") * 2 + lax.axis_index("c")
    base = wid * RPW
    pltpu.sync_copy(idx_hbm.at[pl.ds(base, RPW)], idx_v)
    for c in range(NBUF - 1):
        pltpu.async_copy(y_hbm.at[idx_v.at[pl.ds(c * CH, CH)]],
                         rows_v.at[c], sem_g)
    last_waited_store = -1
    for c in range(NCHUNK):
        pltpu.make_async_copy(y_hbm.at[idx_v.at[pl.ds(c * CH, CH)]],
                              rows_v.at[c % NBUF], sem_g).wait()
        if c + NBUF - 1 < NCHUNK:
            if c >= 1:
                pltpu.make_async_copy(
                    rows_v.at[(c - 1) % NBUF],
                    out_hbm.at[pl.ds(base + (c - 1) * CH, CH)],
                    sem_s).wait()
                last_waited_store = c - 1
            pltpu.async_copy(
                y_hbm.at[idx_v.at[pl.ds((c + NBUF - 1) * CH, CH)]],
                rows_v.at[(c + NBUF - 1) % NBUF], sem_g)
        pltpu.async_copy(rows_v.at[c % NBUF],
                         out_hbm.at[pl.ds(base + c * CH, CH)], sem_s)
    for c in range(last_waited_store + 1, NCHUNK):
        pltpu.make_async_copy(rows_v.at[c % NBUF],
                              out_hbm.at[pl.ds(base + c * CH, CH)],
                              sem_s).wait()


def _sc_call(y, idx_flat):
    mesh = plsc.VectorSubcoreMesh(core_axis_name="c", subcore_axis_name="s")
    f = pl.kernel(
        _sc_body,
        out_type=jax.ShapeDtypeStruct((B * T, DO), jnp.float32),
        mesh=mesh,
        scratch_types=[
            pltpu.VMEM((RPW,), jnp.int32),
            pltpu.VMEM((NBUF, CH, DO), jnp.float32),
            pltpu.SemaphoreType.DMA,
            pltpu.SemaphoreType.DMA,
        ],
    )
    return f(y, idx_flat)


def kernel(pooled_vectors, pooling_indices, target_length, W):
    x_flat = pooled_vectors.reshape(R, DI)
    pi2d = pooling_indices.reshape(1, P).astype(jnp.int32)
    row_idx = _map_call(pi2d)
    y = _mm_call(x_flat, W)
    out = _sc_call(y, row_idx.reshape(B * T))
    return out.reshape(B, T, DO)


# speedup vs baseline: 1.2893x; 1.0021x over previous
import functools

import jax
import jax.numpy as jnp
from jax import lax
from jax.experimental import pallas as pl
from jax.experimental.pallas import tpu as pltpu
from jax.experimental.pallas import tpu_sc as plsc

B = 4
P = 512
T = 2048
DI = 1024
DO = 1024
L = 16
R = B * P
YROWS = (L + 1) * R


def _map_body(pi_ref, idx_ref):
    pi = pi_ref[0, :]
    piB = jnp.broadcast_to(pi[None, :], (T, P))
    t2 = lax.broadcasted_iota(jnp.int32, (T, P), 0)
    lt = piB < t2
    idx = jnp.sum(lt.astype(jnp.int32), axis=1)
    prev = jnp.max(jnp.where(lt, piB, -1), axis=1)
    t1 = lax.iota(jnp.int32, T)
    pos = jnp.minimum(t1 - 1 - prev, L - 1)
    valid = idx < P
    l_eff = jnp.where(valid, pos, L)
    j_eff = jnp.where(valid, idx, 0)
    rows = l_eff * R + j_eff
    boff = lax.broadcasted_iota(jnp.int32, (B, T), 0) * P
    idx_ref[...] = rows[None, :] + boff


def _map_call(pi2d):
    return pl.pallas_call(
        _map_body,
        out_shape=jax.ShapeDtypeStruct((B, T), jnp.int32),
    )(pi2d)


def _mm_body(x_ref, w_ref, y_ref):
    l = pl.program_id(0)

    @pl.when(l < L)
    def _matmul():
        y_ref[...] = lax.dot_general(
            x_ref[...], w_ref[0],
            dimension_numbers=(((1,), (1,)), ((), ())),
            preferred_element_type=jnp.float32,
        )

    @pl.when(l == L)
    def _zeros():
        y_ref[...] = jnp.zeros_like(y_ref)


def _mm_call(x_flat, W):
    return pl.pallas_call(
        _mm_body,
        grid=(L + 1,),
        in_specs=[
            pl.BlockSpec((R, DI), lambda l: (0, 0)),
            pl.BlockSpec((1, DO, DI), lambda l: (jnp.minimum(l, L - 1), 0, 0)),
        ],
        out_specs=pl.BlockSpec((R, DO), lambda l: (l, 0)),
        out_shape=jax.ShapeDtypeStruct((YROWS, DO), jnp.float32),
    )(x_flat, W)


NW = 32
RPW = (B * T) // NW
CH = 16
NCHUNK = RPW // CH


NBUF = 6


def _sc_body(y_hbm, idx_hbm, out_hbm, idx_v, rows_v, sem_g, sem_s):
    wid = lax.axis_index("s") * 2 + lax.axis_index("c")
    base = wid * RPW
    pltpu.sync_copy(idx_hbm.at[pl.ds(base, RPW)], idx_v)
    for c in range(NBUF - 1):
        pltpu.async_copy(y_hbm.at[idx_v.at[pl.ds(c * CH, CH)]],
                         rows_v.at[c], sem_g)
    last_waited_store = -1
    for c in range(NCHUNK):
        pltpu.make_async_copy(y_hbm.at[idx_v.at[pl.ds(c * CH, CH)]],
                              rows_v.at[c % NBUF], sem_g).wait()
        if c + NBUF - 1 < NCHUNK:
            if c >= 1:
                pltpu.make_async_copy(
                    rows_v.at[(c - 1) % NBUF],
                    out_hbm.at[pl.ds(base + (c - 1) * CH, CH)],
                    sem_s).wait()
                last_waited_store = c - 1
            pltpu.async_copy(
                y_hbm.at[idx_v.at[pl.ds((c + NBUF - 1) * CH, CH)]],
                rows_v.at[(c + NBUF - 1) % NBUF], sem_g)
        pltpu.async_copy(rows_v.at[c % NBUF],
                         out_hbm.at[pl.ds(base + c * CH, CH)], sem_s)
    for c in range(last_waited_store + 1, NCHUNK):
        pltpu.make_async_copy(rows_v.at[c % NBUF],
                              out_hbm.at[pl.ds(base + c * CH, CH)],
                              sem_s).wait()


def _sc_call(y, idx_flat):
    mesh = plsc.VectorSubcoreMesh(core_axis_name="c", subcore_axis_name="s")
    f = pl.kernel(
        _sc_body,
        out_type=jax.ShapeDtypeStruct((B * T, DO), jnp.float32),
        mesh=mesh,
        scratch_types=[
            pltpu.VMEM((RPW,), jnp.int32),
            pltpu.VMEM((NBUF, CH, DO), jnp.float32),
            pltpu.SemaphoreType.DMA,
            pltpu.SemaphoreType.DMA,
        ],
    )
    return f(y, idx_flat)


def kernel(pooled_vectors, pooling_indices, target_length, W):
    x_flat = pooled_vectors.reshape(R, DI)
    pi2d = pooling_indices.reshape(1, P).astype(jnp.int32)
    row_idx = _map_call(pi2d)
    y = _mm_call(x_flat, W)
    out = _sc_call(y, row_idx.reshape(B * T))
    return out.reshape(B, T, DO)


# speedup vs baseline: 1.3068x; 1.0135x over previous
import functools

import jax
import jax.numpy as jnp
from jax import lax
from jax.experimental import pallas as pl
from jax.experimental.pallas import tpu as pltpu
from jax.experimental.pallas import tpu_sc as plsc

B = 4
P = 512
T = 2048
DI = 1024
DO = 1024
L = 16
R = B * P
YROWS = (L + 1) * R


def _map_body(pi_ref, idx_ref):
    pi = pi_ref[0, :]
    piB = jnp.broadcast_to(pi[None, :], (T, P))
    t2 = lax.broadcasted_iota(jnp.int32, (T, P), 0)
    lt = piB < t2
    idx = jnp.sum(lt.astype(jnp.int32), axis=1)
    prev = jnp.max(jnp.where(lt, piB, -1), axis=1)
    t1 = lax.iota(jnp.int32, T)
    pos = jnp.minimum(t1 - 1 - prev, L - 1)
    valid = idx < P
    l_eff = jnp.where(valid, pos, L)
    j_eff = jnp.where(valid, idx, 0)
    rows = l_eff * R + j_eff
    boff = lax.broadcasted_iota(jnp.int32, (B, T), 0) * P
    idx_ref[...] = rows[None, :] + boff


def _map_call(pi2d):
    return pl.pallas_call(
        _map_body,
        out_shape=jax.ShapeDtypeStruct((B, T), jnp.int32),
    )(pi2d)


def _mm_body(x_ref, w_ref, y_ref):
    p = pl.program_id(0)

    @pl.when(p > 0)
    def _matmul():
        y_ref[...] = lax.dot_general(
            x_ref[...], w_ref[0],
            dimension_numbers=(((1,), (1,)), ((), ())),
            preferred_element_type=jnp.float32,
        )

    @pl.when(p == 0)
    def _zeros():
        y_ref[...] = jnp.zeros_like(y_ref)


def _mm_call(x_flat, W):
    return pl.pallas_call(
        _mm_body,
        grid=(L + 1,),
        in_specs=[
            pl.BlockSpec((R, DI), lambda p: (0, 0)),
            pl.BlockSpec((1, DO, DI), lambda p: (jnp.maximum(p - 1, 0), 0, 0)),
        ],
        out_specs=pl.BlockSpec((R, DO), lambda p: (jnp.where(p == 0, L, p - 1), 0)),
        out_shape=jax.ShapeDtypeStruct((YROWS, DO), jnp.float32),
    )(x_flat, W)


NW = 32
RPW = (B * T) // NW
CH = 16
NCHUNK = RPW // CH


NBUF = 6


def _sc_body(y_hbm, idx_hbm, out_hbm, idx_v, rows_v, sem_g, sem_s):
    wid = lax.axis_index("s") * 2 + lax.axis_index("c")
    base = wid * RPW
    pltpu.sync_copy(idx_hbm.at[pl.ds(base, RPW)], idx_v)
    for c in range(NBUF - 1):
        pltpu.async_copy(y_hbm.at[idx_v.at[pl.ds(c * CH, CH)]],
                         rows_v.at[c], sem_g)
    last_waited_store = -1
    for c in range(NCHUNK):
        pltpu.make_async_copy(y_hbm.at[idx_v.at[pl.ds(c * CH, CH)]],
                              rows_v.at[c % NBUF], sem_g).wait()
        if c + NBUF - 1 < NCHUNK:
            if c >= 1:
                pltpu.make_async_copy(
                    rows_v.at[(c - 1) % NBUF],
                    out_hbm.at[pl.ds(base + (c - 1) * CH, CH)],
                    sem_s).wait()
                last_waited_store = c - 1
            pltpu.async_copy(
                y_hbm.at[idx_v.at[pl.ds((c + NBUF - 1) * CH, CH)]],
                rows_v.at[(c + NBUF - 1) % NBUF], sem_g)
        pltpu.async_copy(rows_v.at[c % NBUF],
                         out_hbm.at[pl.ds(base + c * CH, CH)], sem_s)
    for c in range(last_waited_store + 1, NCHUNK):
        pltpu.make_async_copy(rows_v.at[c % NBUF],
                              out_hbm.at[pl.ds(base + c * CH, CH)],
                              sem_s).wait()


def _sc_call(y, idx_flat):
    mesh = plsc.VectorSubcoreMesh(core_axis_name="c", subcore_axis_name="s")
    f = pl.kernel(
        _sc_body,
        out_type=jax.ShapeDtypeStruct((B * T, DO), jnp.float32),
        mesh=mesh,
        scratch_types=[
            pltpu.VMEM((RPW,), jnp.int32),
            pltpu.VMEM((NBUF, CH, DO), jnp.float32),
            pltpu.SemaphoreType.DMA,
            pltpu.SemaphoreType.DMA,
        ],
    )
    return f(y, idx_flat)


def kernel(pooled_vectors, pooling_indices, target_length, W):
    x_flat = pooled_vectors.reshape(R, DI)
    pi2d = pooling_indices.reshape(1, P).astype(jnp.int32)
    row_idx = _map_call(pi2d)
    y = _mm_call(x_flat, W)
    out = _sc_call(y, row_idx.reshape(B * T))
    return out.reshape(B, T, DO)


# speedup vs baseline: 1.3218x; 1.0115x over previous
import functools

import jax
import jax.numpy as jnp
from jax import lax
from jax.experimental import pallas as pl
from jax.experimental.pallas import tpu as pltpu
from jax.experimental.pallas import tpu_sc as plsc

B = 4
P = 512
T = 2048
DI = 1024
DO = 1024
L = 16
R = B * P
YROWS = (L + 1) * R


def _map_body(pi_ref, idx_ref):
    pi = pi_ref[0, :]
    piB = jnp.broadcast_to(pi[None, :], (T, P))
    t2 = lax.broadcasted_iota(jnp.int32, (T, P), 0)
    lt = piB < t2
    idx = jnp.sum(lt.astype(jnp.int32), axis=1)
    prev = jnp.max(jnp.where(lt, piB, -1), axis=1)
    t1 = lax.iota(jnp.int32, T)
    pos = jnp.minimum(t1 - 1 - prev, L - 1)
    valid = idx < P
    l_eff = jnp.where(valid, pos, L)
    j_eff = jnp.where(valid, idx, 0)
    rows = l_eff * R + j_eff
    boff = lax.broadcasted_iota(jnp.int32, (B, T), 0) * P
    idx_ref[...] = rows[None, :] + boff


def _map_call(pi2d):
    return pl.pallas_call(
        _map_body,
        out_shape=jax.ShapeDtypeStruct((B, T), jnp.int32),
    )(pi2d)


def _mm_body(pi_ref, x_ref, w_ref, y_ref, idx_ref):
    p = pl.program_id(0)

    @pl.when(p > 0)
    def _matmul():
        y_ref[...] = lax.dot_general(
            x_ref[...], w_ref[0],
            dimension_numbers=(((1,), (1,)), ((), ())),
            preferred_element_type=jnp.float32,
        )

    @pl.when(p == 0)
    def _zeros():
        y_ref[...] = jnp.zeros_like(y_ref)
        _map_body(pi_ref, idx_ref)


def _mm_call(pi2d, x_flat, W):
    return pl.pallas_call(
        _mm_body,
        grid=(L + 1,),
        in_specs=[
            pl.BlockSpec((1, P), lambda p: (0, 0)),
            pl.BlockSpec((R, DI), lambda p: (0, 0)),
            pl.BlockSpec((1, DO, DI), lambda p: (jnp.maximum(p - 1, 0), 0, 0)),
        ],
        out_specs=[
            pl.BlockSpec((R, DO), lambda p: (jnp.where(p == 0, L, p - 1), 0)),
            pl.BlockSpec((B, T), lambda p: (0, 0)),
        ],
        out_shape=[
            jax.ShapeDtypeStruct((YROWS, DO), jnp.float32),
            jax.ShapeDtypeStruct((B, T), jnp.int32),
        ],
    )(pi2d, x_flat, W)


NW = 32
RPW = (B * T) // NW
CH = 16
NCHUNK = RPW // CH


NBUF = 6


def _sc_body(y_hbm, idx_hbm, out_hbm, idx_v, rows_v, sem_g, sem_s):
    wid = lax.axis_index("s") * 2 + lax.axis_index("c")
    base = wid * RPW
    pltpu.sync_copy(idx_hbm.at[pl.ds(base, RPW)], idx_v)
    for c in range(NBUF - 1):
        pltpu.async_copy(y_hbm.at[idx_v.at[pl.ds(c * CH, CH)]],
                         rows_v.at[c], sem_g)
    last_waited_store = -1
    for c in range(NCHUNK):
        pltpu.make_async_copy(y_hbm.at[idx_v.at[pl.ds(c * CH, CH)]],
                              rows_v.at[c % NBUF], sem_g).wait()
        if c + NBUF - 1 < NCHUNK:
            if c >= 1:
                pltpu.make_async_copy(
                    rows_v.at[(c - 1) % NBUF],
                    out_hbm.at[pl.ds(base + (c - 1) * CH, CH)],
                    sem_s).wait()
                last_waited_store = c - 1
            pltpu.async_copy(
                y_hbm.at[idx_v.at[pl.ds((c + NBUF - 1) * CH, CH)]],
                rows_v.at[(c + NBUF - 1) % NBUF], sem_g)
        pltpu.async_copy(rows_v.at[c % NBUF],
                         out_hbm.at[pl.ds(base + c * CH, CH)], sem_s)
    for c in range(last_waited_store + 1, NCHUNK):
        pltpu.make_async_copy(rows_v.at[c % NBUF],
                              out_hbm.at[pl.ds(base + c * CH, CH)],
                              sem_s).wait()


def _sc_call(y, idx_flat):
    mesh = plsc.VectorSubcoreMesh(core_axis_name="c", subcore_axis_name="s")
    f = pl.kernel(
        _sc_body,
        out_type=jax.ShapeDtypeStruct((B * T, DO), jnp.float32),
        mesh=mesh,
        scratch_types=[
            pltpu.VMEM((RPW,), jnp.int32),
            pltpu.VMEM((NBUF, CH, DO), jnp.float32),
            pltpu.SemaphoreType.DMA,
            pltpu.SemaphoreType.DMA,
        ],
    )
    return f(y, idx_flat)


def kernel(pooled_vectors, pooling_indices, target_length, W):
    x_flat = pooled_vectors.reshape(R, DI)
    pi2d = pooling_indices.reshape(1, P).astype(jnp.int32)
    y, row_idx = _mm_call(pi2d, x_flat, W)
    out = _sc_call(y, row_idx.reshape(B * T))
    return out.reshape(B, T, DO)


# speedup vs baseline: 1.3220x; 1.0001x over previous
import jax
import jax.numpy as jnp
from jax import lax
from jax.experimental import pallas as pl
from jax.experimental.pallas import tpu as pltpu
from jax.experimental.pallas import tpu_sc as plsc

B = 4
P = 512
T = 2048
DI = 1024
DO = 1024
L = 16
R = B * P
YROWS = (L + 1) * R


def _map_body(pi_ref, idx_ref):
    pi = pi_ref[0, :]
    piB = jnp.broadcast_to(pi[None, :], (T, P))
    t2 = lax.broadcasted_iota(jnp.int32, (T, P), 0)
    lt = piB < t2
    idx = jnp.sum(lt.astype(jnp.int32), axis=1)
    prev = jnp.max(jnp.where(lt, piB, -1), axis=1)
    t1 = lax.iota(jnp.int32, T)
    pos = jnp.minimum(t1 - 1 - prev, L - 1)
    valid = idx < P
    l_eff = jnp.where(valid, pos, L)
    j_eff = jnp.where(valid, idx, 0)
    rows = l_eff * R + j_eff
    boff = lax.broadcasted_iota(jnp.int32, (B, T), 0) * P
    idx_ref[...] = rows[None, :] + boff


def _mm_body(pi_ref, x_ref, w_ref, y_ref, idx_ref):
    p = pl.program_id(0)

    @pl.when(p > 0)
    def _matmul():
        y_ref[...] = lax.dot_general(
            x_ref[...], w_ref[0],
            dimension_numbers=(((1,), (1,)), ((), ())),
            preferred_element_type=jnp.float32,
        )

    @pl.when(p == 0)
    def _zeros():
        y_ref[...] = jnp.zeros_like(y_ref)
        _map_body(pi_ref, idx_ref)


def _mm_call(pi2d, x_flat, W):
    return pl.pallas_call(
        _mm_body,
        grid=(L + 1,),
        in_specs=[
            pl.BlockSpec((1, P), lambda p: (0, 0)),
            pl.BlockSpec((R, DI), lambda p: (0, 0)),
            pl.BlockSpec((1, DO, DI), lambda p: (jnp.maximum(p - 1, 0), 0, 0)),
        ],
        out_specs=[
            pl.BlockSpec((R, DO), lambda p: (jnp.where(p == 0, L, p - 1), 0)),
            pl.BlockSpec((B, T), lambda p: (0, 0)),
        ],
        out_shape=[
            jax.ShapeDtypeStruct((YROWS, DO), jnp.float32),
            jax.ShapeDtypeStruct((B, T), jnp.int32),
        ],
    )(pi2d, x_flat, W)


NW = 32
RPW = (B * T) // NW
CH = 16
NCHUNK = RPW // CH


NBUF = 6


def _sc_body(y_hbm, idx_hbm, out_hbm, idx_v, rows_v, sem_g, sem_s):
    wid = lax.axis_index("s") * 2 + lax.axis_index("c")
    base = wid * RPW
    pltpu.sync_copy(idx_hbm.at[pl.ds(base, RPW)], idx_v)
    for c in range(NBUF - 1):
        pltpu.async_copy(y_hbm.at[idx_v.at[pl.ds(c * CH, CH)]],
                         rows_v.at[c], sem_g)
    last_waited_store = -1
    for c in range(NCHUNK):
        pltpu.make_async_copy(y_hbm.at[idx_v.at[pl.ds(c * CH, CH)]],
                              rows_v.at[c % NBUF], sem_g).wait()
        if c + NBUF - 1 < NCHUNK:
            if c >= 1:
                pltpu.make_async_copy(
                    rows_v.at[(c - 1) % NBUF],
                    out_hbm.at[pl.ds(base + (c - 1) * CH, CH)],
                    sem_s).wait()
                last_waited_store = c - 1
            pltpu.async_copy(
                y_hbm.at[idx_v.at[pl.ds((c + NBUF - 1) * CH, CH)]],
                rows_v.at[(c + NBUF - 1) % NBUF], sem_g)
        pltpu.async_copy(rows_v.at[c % NBUF],
                         out_hbm.at[pl.ds(base + c * CH, CH)], sem_s)
    for c in range(last_waited_store + 1, NCHUNK):
        pltpu.make_async_copy(rows_v.at[c % NBUF],
                              out_hbm.at[pl.ds(base + c * CH, CH)],
                              sem_s).wait()


def _sc_call(y, idx_flat):
    mesh = plsc.VectorSubcoreMesh(core_axis_name="c", subcore_axis_name="s")
    f = pl.kernel(
        _sc_body,
        out_type=jax.ShapeDtypeStruct((B * T, DO), jnp.float32),
        mesh=mesh,
        scratch_types=[
            pltpu.VMEM((RPW,), jnp.int32),
            pltpu.VMEM((NBUF, CH, DO), jnp.float32),
            pltpu.SemaphoreType.DMA,
            pltpu.SemaphoreType.DMA,
        ],
    )
    return f(y, idx_flat)


def kernel(pooled_vectors, pooling_indices, target_length, W):
    x_flat = pooled_vectors.reshape(R, DI)
    pi2d = pooling_indices.reshape(1, P).astype(jnp.int32)
    y, row_idx = _mm_call(pi2d, x_flat, W)
    out = _sc_call(y, row_idx.reshape(B * T))
    return out.reshape(B, T, DO)
